# baseline (speedup 1.0000x reference)
import jax
import jax.numpy as jnp
from jax.experimental import pallas as pl
from jax.experimental.pallas import tpu as pltpu

_BM = 8
_NBUF = 8


def _mm_kernel(x_hbm, wt_ref, b_ref, o_ref, x_bufs, x_sems):
    B = o_ref.shape[0]
    nchunks = B // _BM

    def start_copy(c, slot):
        pltpu.make_async_copy(
            x_hbm.at[pl.ds(c * _BM, _BM), :],
            x_bufs.at[slot],
            x_sems.at[slot],
        ).start()

    wt = wt_ref[...]
    bias = b_ref[...]

    def process(c, slot):
        pltpu.make_async_copy(
            x_hbm.at[pl.ds(0, _BM), :],
            x_bufs.at[slot],
            x_sems.at[slot],
        ).wait()
        x = x_bufs[slot].astype(jnp.bfloat16)
        part = jax.lax.dot_general(
            x,
            wt,
            dimension_numbers=(((1,), (1,)), ((), ())),
            preferred_element_type=jnp.float32,
        )
        o_ref[pl.ds(c * _BM, _BM), :] = part + bias

    for c in range(_NBUF):
        start_copy(c, c)

    n_main = (nchunks - _NBUF) // _NBUF * _NBUF

    def body(step, carry):
        base = step * _NBUF
        for i in range(_NBUF):
            process(base + i, i)
            start_copy(base + i + _NBUF, i)
        return carry

    jax.lax.fori_loop(0, n_main // _NBUF, body, None)

    for c in range(n_main, nchunks):
        process(c, c % _NBUF)


def kernel(X, W, b):
    B, K = X.shape
    _, N = W.shape
    wt16 = W.T.astype(jnp.bfloat16)
    b2 = b.reshape(1, N)
    return pl.pallas_call(
        _mm_kernel,
        in_specs=[
            pl.BlockSpec(memory_space=pltpu.MemorySpace.HBM),
            pl.BlockSpec((N, K), lambda: (0, 0)),
            pl.BlockSpec((1, N), lambda: (0, 0)),
        ],
        out_specs=pl.BlockSpec((B, N), lambda: (0, 0)),
        out_shape=jax.ShapeDtypeStruct((B, N), jnp.float32),
        scratch_shapes=[
            pltpu.VMEM((_NBUF, _BM, K), jnp.float32),
            pltpu.SemaphoreType.DMA((_NBUF,)),
        ],
    )(X, wt16, b2)


# speedup vs baseline: 1.5947x; 1.5947x over previous
import jax
import jax.numpy as jnp
from jax.experimental import pallas as pl
from jax.experimental.pallas import tpu as pltpu

_BK = 1024
_NBUF = 8
_TB = 2048


def _main_kernel(x_hbm, w_hbm, b_ref, o_ref, x_bufs, w_bufs, x_sems, w_sems):
    K_cov = (x_hbm.shape[1] // _TB) * _TB
    nfull = K_cov // _BK

    def start_copies(c, slot):
        pltpu.make_async_copy(
            x_hbm.at[:, pl.ds(c * _BK, _BK)],
            x_bufs.at[slot],
            x_sems.at[slot],
        ).start()
        pltpu.make_async_copy(
            w_hbm.at[pl.ds(c * _BK, _BK), :],
            w_bufs.at[slot],
            w_sems.at[slot],
        ).start()

    def process(c, slot):
        pltpu.make_async_copy(
            x_hbm.at[:, pl.ds(0, _BK)],
            x_bufs.at[slot],
            x_sems.at[slot],
        ).wait()
        pltpu.make_async_copy(
            w_hbm.at[pl.ds(0, _BK), :],
            w_bufs.at[slot],
            w_sems.at[slot],
        ).wait()
        x = x_bufs[slot].astype(jnp.bfloat16)
        w = w_bufs[slot].astype(jnp.bfloat16)
        o_ref[...] += jax.lax.dot(x, w, preferred_element_type=jnp.float32)

    o_ref[...] = jnp.broadcast_to(b_ref[...], o_ref.shape)

    for c in range(_NBUF):
        start_copies(c, c)

    n_main = (nfull - _NBUF) // _NBUF * _NBUF

    def body(step, carry):
        base = step * _NBUF
        for i in range(_NBUF):
            process(base + i, i)
            start_copies(base + i + _NBUF, i)
        return carry

    jax.lax.fori_loop(0, n_main // _NBUF, body, None)

    for c in range(n_main, nfull):
        process(c, c % _NBUF)


def _tail_kernel(x_ref, w_ref, o_ref, *, valid):
    x = x_ref[...]
    w = w_ref[...]
    cols = jax.lax.broadcasted_iota(jnp.int32, x.shape, 1)
    rows = jax.lax.broadcasted_iota(jnp.int32, w.shape, 0)
    x = jnp.where(cols < valid, x, 0.0).astype(jnp.bfloat16)
    w = jnp.where(rows < valid, w, 0.0).astype(jnp.bfloat16)
    o_ref[...] = jax.lax.dot(x, w, preferred_element_type=jnp.float32)


def kernel(X, W, b):
    import functools

    B, K = X.shape
    _, N = W.shape
    b2 = b.reshape(1, N)
    k_cov = (K // _TB) * _TB
    tail_blk = k_cov // _TB

    main = pl.pallas_call(
        _main_kernel,
        in_specs=[
            pl.BlockSpec(memory_space=pltpu.MemorySpace.HBM),
            pl.BlockSpec(memory_space=pltpu.MemorySpace.HBM),
            pl.BlockSpec((1, N), lambda: (0, 0)),
        ],
        out_specs=pl.BlockSpec((B, N), lambda: (0, 0)),
        out_shape=jax.ShapeDtypeStruct((B, N), jnp.float32),
        scratch_shapes=[
            pltpu.VMEM((_NBUF, B, _BK), jnp.float32),
            pltpu.VMEM((_NBUF, _BK, N), jnp.float32),
            pltpu.SemaphoreType.DMA((_NBUF,)),
            pltpu.SemaphoreType.DMA((_NBUF,)),
        ],
    )(X, W, b2)

    tail = pl.pallas_call(
        functools.partial(_tail_kernel, valid=K - k_cov),
        grid=(1,),
        in_specs=[
            pl.BlockSpec((B, _TB), lambda i: (0, tail_blk)),
            pl.BlockSpec((_TB, N), lambda i: (tail_blk, 0)),
        ],
        out_specs=pl.BlockSpec((B, N), lambda i: (0, 0)),
        out_shape=jax.ShapeDtypeStruct((B, N), jnp.float32),
    )(X, W)

    return main + tail
